# Initial kernel scaffold; baseline (speedup 1.0000x reference)
import jax
import jax.numpy as jnp
from jax.experimental import pallas as pl


def kernel(x, edge_index, edge_weight, batch, W1_rel, b1_rel, W1_root, W2_rel, b2_rel, W2_root, Wc, bc):
    raise NotImplementedError("write your pallas kernel here")


# speedup vs baseline: 5.3008x; 5.3008x over previous
import dataclasses
import functools

import jax
import jax.numpy as jnp
from jax import lax
from jax.experimental import pallas as pl
from jax.experimental.pallas import tpu as pltpu
from jax.experimental.pallas import tpu_sc as plsc

N = 10000
E = 320000
D = 128
H = 256
H2 = 64
OUT = 10
G = 64

NC = 2
NS = 16
NW = NC * NS
EPT = E // NW
CH = 80
NCHUNK = EPT // CH
ZR = 80
NRCH = N // ZR


DB = 64


def _make_sc_segsum(nt):
  mesh = plsc.VectorSubcoreMesh(core_axis_name="c", subcore_axis_name="s")
  cp = pltpu.CompilerParams()
  if "needs_layout_passes" in pltpu.CompilerParams.__dataclass_fields__:
    cp = dataclasses.replace(cp, needs_layout_passes=False)
  if "use_tc_tiling_on_sc" in pltpu.CompilerParams.__dataclass_fields__:
    cp = dataclasses.replace(cp, use_tc_tiling_on_sc=False)

  @functools.partial(
      pl.kernel,
      out_type=jax.ShapeDtypeStruct((NC, nt, N, DB), jnp.float32),
      mesh=mesh,
      compiler_params=cp,
      scratch_types=[
          pltpu.VMEM((NCHUNK, CH), jnp.int32),
          pltpu.VMEM((NCHUNK, CH), jnp.int32),
          pltpu.VMEM((EPT,), jnp.float32),
          pltpu.VMEM((CH, DB), jnp.float32),
          pltpu.VMEM((ZR, DB), jnp.float32),
          pltpu.VMEM_SHARED((N, DB), jnp.float32),
          pltpu.SemaphoreType.DMA,
      ],
  )
  def sc_segsum(*refs):
    tables = refs[:nt]
    (src_hbm, dst_hbm, ew_hbm, out_hbm,
     src_v, dst_v, ew_v, rows_v, zero_v, acc_sh, sem) = refs[nt:]
    c = lax.axis_index("c")
    s = lax.axis_index("s")
    wid = c * NS + s

    pltpu.sync_copy(src_hbm.at[wid], src_v)
    pltpu.sync_copy(dst_hbm.at[wid], dst_v)
    pltpu.sync_copy(ew_hbm.at[pl.ds(wid * EPT, EPT)], ew_v)

    @pl.loop(0, ZR)
    def _(r):
      @pl.loop(0, DB, step=16)
      def _(cc):
        zero_v[r, pl.ds(cc, 16)] = jnp.zeros((16,), jnp.float32)

    for t in range(nt):
      @pl.loop(0, (NRCH + NS - 1) // NS)
      def _(k):
        rc = k * NS + s

        @pl.when(rc < NRCH)
        def _():
          pltpu.sync_copy(zero_v, acc_sh.at[pl.ds(rc * ZR, ZR)])

      plsc.subcore_barrier()

      @pl.loop(0, NCHUNK)
      def _(j):
        pltpu.async_copy(tables[t].at[src_v.at[j]], rows_v, sem).wait()

        @pl.loop(0, CH)
        def _(e):
          w = plsc.load_gather(ew_v, [jnp.full((16,), j * CH + e, jnp.int32)])
          for cc in range(0, DB, 16):
            rows_v[e, pl.ds(cc, 16)] = rows_v[e, pl.ds(cc, 16)] * w

        pltpu.sync_copy(rows_v, acc_sh.at[dst_v.at[j]], add=True)

      plsc.subcore_barrier()

      @pl.loop(0, (NRCH + NS - 1) // NS)
      def _(k):
        rc = k * NS + s

        @pl.when(rc < NRCH)
        def _():
          sl = pl.ds(rc * ZR, ZR)
          pltpu.sync_copy(acc_sh.at[sl], out_hbm.at[c, t, sl])

      plsc.subcore_barrier()

  return sc_segsum


_sc_segsum_d = _make_sc_segsum(2)
_sc_segsum_h2 = _make_sc_segsum(1)

BN = 1000
NB = N // BN


def _tc_dense1(a00, a01, a10, a11, x, w1relT_t, w1relT_b, b1, w1rootT,
               w2relT, w2rootT, b2):
  def body(a00_ref, a01_ref, a10_ref, a11_ref, x_ref, wrt_ref, wrb_ref,
           b1_ref, wroot_ref, w2r_ref, w2root_ref, b2_ref, p1_ref, hr2_ref):
    agg_t = a00_ref[...] + a10_ref[...]
    agg_b = a01_ref[...] + a11_ref[...]
    h1 = jnp.maximum(
        jnp.dot(agg_t, wrt_ref[...], preferred_element_type=jnp.float32)
        + jnp.dot(agg_b, wrb_ref[...], preferred_element_type=jnp.float32)
        + jnp.dot(x_ref[...], wroot_ref[...], preferred_element_type=jnp.float32)
        + b1_ref[...], 0.0)
    p1_ref[...] = jnp.dot(h1, w2r_ref[...], preferred_element_type=jnp.float32)
    hr2_ref[...] = (
        jnp.dot(h1, w2root_ref[...], preferred_element_type=jnp.float32)
        + b2_ref[...])

  row = lambda i: (i, 0)
  full = lambda i: (0, 0)
  return pl.pallas_call(
      body,
      grid=(NB,),
      in_specs=[
          pl.BlockSpec((BN, DB), row),
          pl.BlockSpec((BN, DB), row),
          pl.BlockSpec((BN, DB), row),
          pl.BlockSpec((BN, DB), row),
          pl.BlockSpec((BN, D), row),
          pl.BlockSpec((DB, H), full),
          pl.BlockSpec((DB, H), full),
          pl.BlockSpec((1, H), full),
          pl.BlockSpec((D, H), full),
          pl.BlockSpec((H, H2), full),
          pl.BlockSpec((H, H2), full),
          pl.BlockSpec((1, H2), full),
      ],
      out_specs=[
          pl.BlockSpec((BN, H2), row),
          pl.BlockSpec((BN, H2), row),
      ],
      out_shape=[
          jax.ShapeDtypeStruct((N, H2), jnp.float32),
          jax.ShapeDtypeStruct((N, H2), jnp.float32),
      ],
  )(a00, a01, a10, a11, x, w1relT_t, w1relT_b, b1, w1rootT, w2relT,
    w2rootT, b2)


def _tc_pool(a0, a1, hr2, batch3, wcT, bc):
  def body(a0_ref, a1_ref, hr_ref, b_ref, wc_ref, bc_ref, o_ref,
           sums_ref, cnts_ref):
    i = pl.program_id(0)

    @pl.when(i == 0)
    def _():
      sums_ref[...] = jnp.zeros_like(sums_ref)
      cnts_ref[...] = jnp.zeros_like(cnts_ref)

    h2 = jnp.maximum(a0_ref[...] + a1_ref[...] + hr_ref[...], 0.0)
    bvec = b_ref[0, 0, :]
    onehot = jnp.where(
        bvec[None, :] == lax.broadcasted_iota(jnp.int32, (G, BN), 0),
        1.0, 0.0)
    sums_ref[...] += jnp.dot(onehot, h2, preferred_element_type=jnp.float32)
    cnts_ref[...] += jnp.sum(onehot, axis=1, keepdims=True)

    @pl.when(i == NB - 1)
    def _():
      pooled = sums_ref[...] / jnp.maximum(cnts_ref[...], 1.0)
      o_ref[...] = (
          jnp.dot(pooled, wc_ref[...], preferred_element_type=jnp.float32)
          + bc_ref[...])

  row = lambda i: (i, 0)
  return pl.pallas_call(
      body,
      grid=(NB,),
      in_specs=[
          pl.BlockSpec((BN, H2), row),
          pl.BlockSpec((BN, H2), row),
          pl.BlockSpec((BN, H2), row),
          pl.BlockSpec((1, 1, BN), lambda i: (i, 0, 0)),
          pl.BlockSpec((H2, OUT), lambda i: (0, 0)),
          pl.BlockSpec((1, OUT), lambda i: (0, 0)),
      ],
      out_specs=pl.BlockSpec((G, OUT), lambda i: (0, 0)),
      out_shape=jax.ShapeDtypeStruct((G, OUT), jnp.float32),
      scratch_shapes=[
          pltpu.VMEM((G, H2), jnp.float32),
          pltpu.VMEM((G, 1), jnp.float32),
      ],
  )(a0, a1, hr2, batch3, wcT, bc)


def kernel(x, edge_index, edge_weight, batch, W1_rel, b1_rel, W1_root,
           W2_rel, b2_rel, W2_root, Wc, bc):
  x = x.astype(jnp.float32)
  ew = edge_weight.astype(jnp.float32)
  src = edge_index[0].reshape(NW, NCHUNK, CH)
  dst = edge_index[1].reshape(NW, NCHUNK, CH)
  x_t = x[:, :DB]
  x_b = x[:, DB:]
  w1relT = W1_rel.T

  agg1 = _sc_segsum_d(x_t, x_b, src, dst, ew)
  p1, hr2 = _tc_dense1(
      agg1[0, 0], agg1[0, 1], agg1[1, 0], agg1[1, 1], x,
      w1relT[:DB], w1relT[DB:], b1_rel.reshape(1, H), W1_root.T,
      W2_rel.T, W2_root.T, b2_rel.reshape(1, H2))
  agg2 = _sc_segsum_h2(p1, src, dst, ew)
  out = _tc_pool(agg2[0, 0], agg2[1, 0], hr2,
                 batch.reshape(NB, 1, BN), Wc.T, bc.reshape(1, OUT))
  return out


# speedup vs baseline: 9.6089x; 1.8127x over previous
import dataclasses
import functools

import jax
import jax.numpy as jnp
from jax import lax
from jax.experimental import pallas as pl
from jax.experimental.pallas import tpu as pltpu
from jax.experimental.pallas import tpu_sc as plsc

N = 10000
E = 320000
D = 128
H = 256
H2 = 64
OUT = 10
G = 64

NC = 2
NS = 16
NW = NC * NS
EPT = E // NW
CH = 80
NCHUNK = EPT // CH
ZR = 80
NRCH = N // ZR


DB = 64


def _make_sc_segsum(nt):
  mesh = plsc.VectorSubcoreMesh(core_axis_name="c", subcore_axis_name="s")
  cp = pltpu.CompilerParams()
  if "needs_layout_passes" in pltpu.CompilerParams.__dataclass_fields__:
    cp = dataclasses.replace(cp, needs_layout_passes=False)
  if "use_tc_tiling_on_sc" in pltpu.CompilerParams.__dataclass_fields__:
    cp = dataclasses.replace(cp, use_tc_tiling_on_sc=False)

  @functools.partial(
      pl.kernel,
      out_type=jax.ShapeDtypeStruct((NC, nt, N, DB), jnp.float32),
      mesh=mesh,
      compiler_params=cp,
      scratch_types=[
          pltpu.VMEM((NCHUNK, CH), jnp.int32),
          pltpu.VMEM((NCHUNK, CH), jnp.int32),
          pltpu.VMEM((EPT,), jnp.float32),
          pltpu.VMEM((CH, DB), jnp.float32),
          pltpu.VMEM((CH, DB), jnp.float32),
          pltpu.VMEM((CH, DB), jnp.float32),
          pltpu.VMEM((ZR, DB), jnp.float32),
          pltpu.VMEM_SHARED((N, DB), jnp.float32),
          pltpu.SemaphoreType.DMA,
          pltpu.SemaphoreType.DMA,
          pltpu.SemaphoreType.DMA,
          pltpu.SemaphoreType.DMA,
          pltpu.SemaphoreType.DMA,
          pltpu.SemaphoreType.DMA,
          pltpu.SemaphoreType.DMA,
      ],
  )
  def sc_segsum(*refs):
    tables = refs[:nt]
    (src_hbm, dst_hbm, ew_hbm, out_hbm,
     src_v, dst_v, ew_v, rows0, rows1, rows2, zero_v, acc_sh, sem,
     gs0, gs1, gs2, ss0, ss1, ss2) = refs[nt:]
    rows_bufs = (rows0, rows1, rows2)
    gsems = (gs0, gs1, gs2)
    ssems = (ss0, ss1, ss2)
    c = lax.axis_index("c")
    s = lax.axis_index("s")
    wid = c * NS + s

    pltpu.sync_copy(src_hbm.at[wid], src_v)
    pltpu.sync_copy(dst_hbm.at[wid], dst_v)
    pltpu.sync_copy(ew_hbm.at[pl.ds(wid * EPT, EPT)], ew_v)

    @pl.loop(0, ZR)
    def _(r):
      @pl.loop(0, DB, step=16)
      def _(cc):
        zero_v[r, pl.ds(cc, 16)] = jnp.zeros((16,), jnp.float32)

    for t in range(nt):
      @pl.loop(0, (NRCH + NS - 1) // NS)
      def _(k):
        rc = k * NS + s

        @pl.when(rc < NRCH)
        def _():
          pltpu.sync_copy(zero_v, acc_sh.at[pl.ds(rc * ZR, ZR)])

      plsc.subcore_barrier()

      def gather_of(j, b):
        return pltpu.make_async_copy(
            tables[t].at[src_v.at[j]], rows_bufs[b], gsems[b])

      def scatter_of(j, b):
        return pltpu.make_async_copy(
            rows_bufs[b], acc_sh.at[dst_v.at[j]], ssems[b])

      pltpu.async_copy(tables[t].at[src_v.at[0]], rows_bufs[0], gsems[0])
      pltpu.async_copy(tables[t].at[src_v.at[1]], rows_bufs[1], gsems[1])

      @pl.loop(0, (NCHUNK + 2) // 3)
      def _(k):
        for i in range(3):
          j = k * 3 + i
          rows_v = rows_bufs[i]

          @pl.when(j < NCHUNK)
          def _():
            gather_of(j, i).wait()

            @pl.loop(0, CH)
            def _(e):
              w = plsc.load_gather(
                  ew_v, [jnp.full((16,), j * CH + e, jnp.int32)])
              for cc in range(0, DB, 16):
                rows_v[e, pl.ds(cc, 16)] = rows_v[e, pl.ds(cc, 16)] * w

            pltpu.async_copy(rows_v, acc_sh.at[dst_v.at[j]], ssems[i],
                             add=True)

            @pl.when(j + 2 < NCHUNK)
            def _():
              bn = (i + 2) % 3

              @pl.when(j >= 1)
              def _():
                scatter_of(j - 1, bn).wait()

              pltpu.async_copy(
                  tables[t].at[src_v.at[j + 2]], rows_bufs[bn], gsems[bn])

      for jj in range(NCHUNK - 3, NCHUNK):
        scatter_of(jj, jj % 3).wait()

      plsc.subcore_barrier()

      @pl.loop(0, (NRCH + NS - 1) // NS)
      def _(k):
        rc = k * NS + s

        @pl.when(rc < NRCH)
        def _():
          sl = pl.ds(rc * ZR, ZR)
          pltpu.sync_copy(acc_sh.at[sl], out_hbm.at[c, t, sl])

      plsc.subcore_barrier()

  return sc_segsum


_sc_segsum_d = _make_sc_segsum(2)
_sc_segsum_h2 = _make_sc_segsum(1)

BN = 1000
NB = N // BN


def _tc_dense1(a00, a01, a10, a11, x, w1relT_t, w1relT_b, b1, w1rootT,
               w2relT, w2rootT, b2):
  def body(a00_ref, a01_ref, a10_ref, a11_ref, x_ref, wrt_ref, wrb_ref,
           b1_ref, wroot_ref, w2r_ref, w2root_ref, b2_ref, p1_ref, hr2_ref):
    agg_t = a00_ref[...] + a10_ref[...]
    agg_b = a01_ref[...] + a11_ref[...]
    h1 = jnp.maximum(
        jnp.dot(agg_t, wrt_ref[...], preferred_element_type=jnp.float32)
        + jnp.dot(agg_b, wrb_ref[...], preferred_element_type=jnp.float32)
        + jnp.dot(x_ref[...], wroot_ref[...], preferred_element_type=jnp.float32)
        + b1_ref[...], 0.0)
    p1_ref[...] = jnp.dot(h1, w2r_ref[...], preferred_element_type=jnp.float32)
    hr2_ref[...] = (
        jnp.dot(h1, w2root_ref[...], preferred_element_type=jnp.float32)
        + b2_ref[...])

  row = lambda i: (i, 0)
  full = lambda i: (0, 0)
  return pl.pallas_call(
      body,
      grid=(NB,),
      in_specs=[
          pl.BlockSpec((BN, DB), row),
          pl.BlockSpec((BN, DB), row),
          pl.BlockSpec((BN, DB), row),
          pl.BlockSpec((BN, DB), row),
          pl.BlockSpec((BN, D), row),
          pl.BlockSpec((DB, H), full),
          pl.BlockSpec((DB, H), full),
          pl.BlockSpec((1, H), full),
          pl.BlockSpec((D, H), full),
          pl.BlockSpec((H, H2), full),
          pl.BlockSpec((H, H2), full),
          pl.BlockSpec((1, H2), full),
      ],
      out_specs=[
          pl.BlockSpec((BN, H2), row),
          pl.BlockSpec((BN, H2), row),
      ],
      out_shape=[
          jax.ShapeDtypeStruct((N, H2), jnp.float32),
          jax.ShapeDtypeStruct((N, H2), jnp.float32),
      ],
  )(a00, a01, a10, a11, x, w1relT_t, w1relT_b, b1, w1rootT, w2relT,
    w2rootT, b2)


def _tc_pool(a0, a1, hr2, batch3, wcT, bc):
  def body(a0_ref, a1_ref, hr_ref, b_ref, wc_ref, bc_ref, o_ref,
           sums_ref, cnts_ref):
    i = pl.program_id(0)

    @pl.when(i == 0)
    def _():
      sums_ref[...] = jnp.zeros_like(sums_ref)
      cnts_ref[...] = jnp.zeros_like(cnts_ref)

    h2 = jnp.maximum(a0_ref[...] + a1_ref[...] + hr_ref[...], 0.0)
    bvec = b_ref[0, 0, :]
    onehot = jnp.where(
        bvec[None, :] == lax.broadcasted_iota(jnp.int32, (G, BN), 0),
        1.0, 0.0)
    sums_ref[...] += jnp.dot(onehot, h2, preferred_element_type=jnp.float32)
    cnts_ref[...] += jnp.sum(onehot, axis=1, keepdims=True)

    @pl.when(i == NB - 1)
    def _():
      pooled = sums_ref[...] / jnp.maximum(cnts_ref[...], 1.0)
      o_ref[...] = (
          jnp.dot(pooled, wc_ref[...], preferred_element_type=jnp.float32)
          + bc_ref[...])

  row = lambda i: (i, 0)
  return pl.pallas_call(
      body,
      grid=(NB,),
      in_specs=[
          pl.BlockSpec((BN, H2), row),
          pl.BlockSpec((BN, H2), row),
          pl.BlockSpec((BN, H2), row),
          pl.BlockSpec((1, 1, BN), lambda i: (i, 0, 0)),
          pl.BlockSpec((H2, OUT), lambda i: (0, 0)),
          pl.BlockSpec((1, OUT), lambda i: (0, 0)),
      ],
      out_specs=pl.BlockSpec((G, OUT), lambda i: (0, 0)),
      out_shape=jax.ShapeDtypeStruct((G, OUT), jnp.float32),
      scratch_shapes=[
          pltpu.VMEM((G, H2), jnp.float32),
          pltpu.VMEM((G, 1), jnp.float32),
      ],
  )(a0, a1, hr2, batch3, wcT, bc)


def kernel(x, edge_index, edge_weight, batch, W1_rel, b1_rel, W1_root,
           W2_rel, b2_rel, W2_root, Wc, bc):
  x = x.astype(jnp.float32)
  ew = edge_weight.astype(jnp.float32)
  src = edge_index[0].reshape(NW, NCHUNK, CH)
  dst = edge_index[1].reshape(NW, NCHUNK, CH)
  x_t = x[:, :DB]
  x_b = x[:, DB:]
  w1relT = W1_rel.T

  agg1 = _sc_segsum_d(x_t, x_b, src, dst, ew)
  p1, hr2 = _tc_dense1(
      agg1[0, 0], agg1[0, 1], agg1[1, 0], agg1[1, 1], x,
      w1relT[:DB], w1relT[DB:], b1_rel.reshape(1, H), W1_root.T,
      W2_rel.T, W2_root.T, b2_rel.reshape(1, H2))
  agg2 = _sc_segsum_h2(p1, src, dst, ew)
  out = _tc_pool(agg2[0, 0], agg2[1, 0], hr2,
                 batch.reshape(NB, 1, BN), Wc.T, bc.reshape(1, OUT))
  return out
